# baseline (speedup 1.0000x reference)
import functools

import jax
import jax.numpy as jnp
from jax import lax
from jax.experimental import pallas as pl
from jax.experimental.pallas import tpu as pltpu
from jax.experimental.pallas import tpu_sc as plsc

_NC, _NS, _L = 2, 16, 16
_NW = _NC * _NS


def _lut_kernel(total, species_hbm, conv_hbm, coords_dep_hbm, out_hbm,
                spec_v, conv_v, out_v, sem_c, sem_s):
    del coords_dep_hbm
    chunk = total // _NW
    wid = lax.axis_index("s") * _NC + lax.axis_index("c")
    base = wid * chunk
    quarter = chunk // 4
    cp_c = pltpu.async_copy(conv_hbm, conv_v, sem_c)
    cp_s = pltpu.async_copy(species_hbm.at[pl.ds(base, chunk)], spec_v, sem_s)
    cp_c.wait()
    cp_s.wait()

    copies = []
    for q in range(4):
        lo = q * quarter

        @plsc.parallel_loop(lo, lo + quarter, step=_L, unroll=8)
        def _gather(off):
            idx = spec_v[pl.ds(off, _L)]
            out_v[pl.ds(off, _L)] = plsc.load_gather(conv_v, [idx])

        copies.append(pltpu.async_copy(
            out_v.at[pl.ds(lo, quarter)],
            out_hbm.at[pl.ds(base + lo, quarter)], sem_s))
    for cp in copies:
        cp.wait()


def kernel(species, coordinates, conv_tensor):
    shape = species.shape
    flat = species.reshape(-1)
    total = flat.shape[0]
    chunk = total // _NW
    one = ((conv_tensor[0] | 1) & 1).astype(coordinates.dtype)
    coords_out = coordinates * one
    mesh = plsc.VectorSubcoreMesh(
        core_axis_name="c", subcore_axis_name="s", num_cores=_NC,
        num_subcores=_NS)
    out = pl.kernel(
        functools.partial(_lut_kernel, total),
        out_type=jax.ShapeDtypeStruct((total,), jnp.int32),
        mesh=mesh,
        scratch_types=[
            pltpu.VMEM((chunk,), jnp.int32),
            pltpu.VMEM((conv_tensor.shape[0],), jnp.int32),
            pltpu.VMEM((chunk,), jnp.int32),
            pltpu.SemaphoreType.DMA,
            pltpu.SemaphoreType.DMA,
        ],
        compiler_params=pltpu.CompilerParams(needs_layout_passes=False),
    )(flat, conv_tensor, coords_out)
    return (out.reshape(shape), coords_out)


# speedup vs baseline: 28.3980x; 28.3980x over previous
import functools

import jax
import jax.numpy as jnp
from jax import lax
from jax.experimental import pallas as pl
from jax.experimental.pallas import tpu as pltpu
from jax.experimental.pallas import tpu_sc as plsc

_NC, _NS, _L = 2, 16, 16
_NW = _NC * _NS


def _lut_kernel(total, species_hbm, conv_hbm, out_hbm,
                spec_v, conv_v, out_v, sem_c, sem_s):
    chunk = total // _NW
    wid = lax.axis_index("s") * _NC + lax.axis_index("c")
    base = wid * chunk
    quarter = chunk // 4
    cp_c = pltpu.async_copy(conv_hbm, conv_v, sem_c)
    cp_s = pltpu.async_copy(species_hbm.at[pl.ds(base, chunk)], spec_v, sem_s)
    cp_c.wait()
    cp_s.wait()

    copies = []
    for q in range(4):
        lo = q * quarter

        @plsc.parallel_loop(lo, lo + quarter, step=_L, unroll=8)
        def _gather(off):
            idx = spec_v[pl.ds(off, _L)]
            out_v[pl.ds(off, _L)] = plsc.load_gather(conv_v, [idx])

        copies.append(pltpu.async_copy(
            out_v.at[pl.ds(lo, quarter)],
            out_hbm.at[pl.ds(base + lo, quarter)], sem_s))
    for cp in copies:
        cp.wait()


def kernel(species, coordinates, conv_tensor):
    shape = species.shape
    flat = species.reshape(-1)
    total = flat.shape[0]
    chunk = total // _NW
    one = ((conv_tensor[0] | 1) & 1).astype(coordinates.dtype)
    coords_out = coordinates * one
    mesh = plsc.VectorSubcoreMesh(
        core_axis_name="c", subcore_axis_name="s", num_cores=_NC,
        num_subcores=_NS)
    out = pl.kernel(
        functools.partial(_lut_kernel, total),
        out_type=jax.ShapeDtypeStruct((total,), jnp.int32),
        mesh=mesh,
        scratch_types=[
            pltpu.VMEM((chunk,), jnp.int32),
            pltpu.VMEM((conv_tensor.shape[0],), jnp.int32),
            pltpu.VMEM((chunk,), jnp.int32),
            pltpu.SemaphoreType.DMA,
            pltpu.SemaphoreType.DMA,
        ],
        compiler_params=pltpu.CompilerParams(needs_layout_passes=False),
    )(flat, conv_tensor)
    return (out.reshape(shape), coords_out)
